# Initial kernel scaffold; baseline (speedup 1.0000x reference)
import jax
import jax.numpy as jnp
from jax.experimental import pallas as pl


def kernel(x, table, pos):
    raise NotImplementedError("write your pallas kernel here")


# speedup vs baseline: 1.4920x; 1.4920x over previous
import functools

import jax
import jax.numpy as jnp
from jax import lax
from jax.experimental import pallas as pl
from jax.experimental.pallas import tpu as pltpu
from jax.experimental.pallas import tpu_sc as plsc

N_VOCAB = 1000000
EMBED_DIM = 32
MAX_LEN = 200
BATCH = 4096

NC, NS, L = 2, 16, 16
NW = NC * NS
TOKENS = BATCH * MAX_LEN
PER_W = TOKENS // NW
ROWS_PER_CHUNK = 8
CHUNK = ROWS_PER_CHUNK * MAX_LEN
NCHUNK = PER_W // CHUNK


def _make_emb_kernel():
    mesh = plsc.VectorSubcoreMesh(core_axis_name="c", subcore_axis_name="s")

    @functools.partial(
        pl.kernel,
        out_type=jax.ShapeDtypeStruct((TOKENS, EMBED_DIM), jnp.float32),
        mesh=mesh,
        compiler_params=pltpu.CompilerParams(use_tc_tiling_on_sc=False),
        scratch_types=[
            pltpu.VMEM((CHUNK,), jnp.int32),
            pltpu.VMEM((CHUNK,), jnp.int32),
            pltpu.VMEM((CHUNK, EMBED_DIM), jnp.float32),
            pltpu.VMEM((CHUNK, EMBED_DIM), jnp.float32),
            pltpu.VMEM((MAX_LEN, EMBED_DIM), jnp.float32),
            pltpu.SemaphoreType.DMA,
            pltpu.SemaphoreType.DMA,
            pltpu.SemaphoreType.DMA,
            pltpu.SemaphoreType.DMA,
        ],
    )
    def emb(table_hbm, idx_hbm, pos_hbm, out_hbm, idx_v0, idx_v1,
            rows_v0, rows_v1, pos_v, gsem0, gsem1, ssem0, ssem1):
        wid = lax.axis_index("s") * NC + lax.axis_index("c")
        base = wid * PER_W
        idx_v = (idx_v0, idx_v1)
        rows_v = (rows_v0, rows_v1)
        gsem = (gsem0, gsem1)
        ssem = (ssem0, ssem1)

        pltpu.sync_copy(pos_hbm, pos_v)

        def add_pos(buf):
            rows = rows_v[buf]
            def body(t, _):
                p0 = pos_v[t, pl.ds(0, L)]
                p1 = pos_v[t, pl.ds(L, L)]
                for g in range(ROWS_PER_CHUNK):
                    r = g * MAX_LEN + t
                    rows[r, pl.ds(0, L)] = rows[r, pl.ds(0, L)] + p0
                    rows[r, pl.ds(L, L)] = rows[r, pl.ds(L, L)] + p1
                return _
            lax.fori_loop(0, MAX_LEN, body, None)

        gather_h = {}
        scatter_h = {}

        pltpu.sync_copy(idx_hbm.at[pl.ds(base, CHUNK)], idx_v[0])
        gather_h[0] = pltpu.async_copy(
            table_hbm.at[idx_v[0]], rows_v[0], gsem[0])

        for k in range(NCHUNK):
            buf = k & 1
            nbuf = 1 - buf
            if k + 1 < NCHUNK:
                pltpu.sync_copy(
                    idx_hbm.at[pl.ds(base + (k + 1) * CHUNK, CHUNK)],
                    idx_v[nbuf])
                if k >= 1:
                    scatter_h[k - 1].wait()
                gather_h[k + 1] = pltpu.async_copy(
                    table_hbm.at[idx_v[nbuf]], rows_v[nbuf], gsem[nbuf])
            gather_h[k].wait()
            add_pos(buf)
            scatter_h[k] = pltpu.async_copy(
                rows_v[buf],
                out_hbm.at[pl.ds(base + k * CHUNK, CHUNK)],
                ssem[buf])
        scatter_h[NCHUNK - 2].wait()
        scatter_h[NCHUNK - 1].wait()

    return emb


_emb_kernel = jax.jit(_make_emb_kernel())


def kernel(x, table, pos):
    xf = x.reshape(-1).astype(jnp.int32)
    out = _emb_kernel(table, xf, pos)
    return out.reshape(x.shape[0], x.shape[1], EMBED_DIM)
